# baseline (speedup 1.0000x reference)
import functools

import jax
import jax.numpy as jnp
from jax import lax
from jax.experimental import pallas as pl
from jax.experimental.pallas import tpu as pltpu
from jax.experimental.pallas import tpu_sc as plsc

NUM_CODES = 8192
DIM = 256
NUM_TOKENS = 8192
TB = 256
NUM_TB = NUM_TOKENS // TB
CW = 128


def _dist_argmin_onehot_kernel(z_ref, cb_ref, idx_ref, oh_ref, csq_ref,
                               aux_ref):
    @pl.when(pl.program_id(0) == 0)
    def _():
        cb0 = cb_ref[...]
        csq_ref[...] = jnp.sum(cb0 * cb0, axis=1)[None, :]
        idsr = lax.broadcasted_iota(
            jnp.int32, (1, NUM_CODES), 1).astype(jnp.float32)
        aux_ref[...] = jnp.concatenate(
            [jnp.ones((1, NUM_CODES), jnp.float32), idsr], axis=0)

    z = z_ref[...]
    zsq = jnp.sum(z * z, axis=1, keepdims=True)
    scores = lax.dot_general(
        z, cb_ref[...], (((1,), (1,)), ((), ())),
        preferred_element_type=jnp.float32)
    d = (zsq + csq_ref[...]) - 2.0 * scores
    mval = jnp.min(d, axis=1, keepdims=True)
    oh = (d == mval).astype(jnp.float32)
    oh_ref[...] = oh
    cnt_idx = lax.dot_general(
        oh, aux_ref[...], (((1,), (1,)), ((), ())),
        preferred_element_type=jnp.float32)
    idx_ref[0, 0, :] = cnt_idx[:, 1].astype(jnp.int32)

    @pl.when(jnp.max(cnt_idx[:, 0]) > 1.5)
    def _():
        ids = aux_ref[1:2, :]
        idxf = jnp.min(
            jnp.where(d == mval, ids, float(NUM_CODES)), axis=1)
        idx_ref[0, 0, :] = idxf.astype(jnp.int32)
        oh_ref[...] = (ids == idxf[:, None]).astype(jnp.float32)


def _dist_argmin_onehot(z_flat, codebook):
    return pl.pallas_call(
        _dist_argmin_onehot_kernel,
        grid=(NUM_TB,),
        in_specs=[
            pl.BlockSpec((TB, DIM), lambda i: (i, 0)),
            pl.BlockSpec((NUM_CODES, DIM), lambda i: (0, 0)),
        ],
        out_specs=[
            pl.BlockSpec((1, 1, TB), lambda i: (i, 0, 0)),
            pl.BlockSpec((TB, NUM_CODES), lambda i: (i, 0)),
        ],
        out_shape=[
            jax.ShapeDtypeStruct((NUM_TB, 1, TB), jnp.int32),
            jax.ShapeDtypeStruct((NUM_TOKENS, NUM_CODES), jnp.float32),
        ],
        scratch_shapes=[pltpu.VMEM((1, NUM_CODES), jnp.float32),
                        pltpu.VMEM((2, NUM_CODES), jnp.float32)],
    )(z_flat, codebook)


def _sc_gather(codebook, idx):
    info = plsc.get_sparse_core_info()
    nc, ns = info.num_cores, info.num_subcores
    nw = nc * ns
    b_per_w = NUM_TOKENS // nw
    mesh = plsc.VectorSubcoreMesh(core_axis_name="c", subcore_axis_name="s")

    @functools.partial(
        pl.kernel, mesh=mesh,
        out_type=jax.ShapeDtypeStruct((NUM_TOKENS, DIM), jnp.float32),
        scratch_types=[
            pltpu.VMEM((b_per_w,), jnp.int32),
            pltpu.VMEM((b_per_w, DIM), jnp.float32),
            pltpu.SemaphoreType.DMA,
        ],
    )
    def gather_k(cb_hbm, idx_hbm, out_hbm, idx_v, rows_v, sem):
        wid = lax.axis_index("s") * nc + lax.axis_index("c")
        base = wid * b_per_w
        pltpu.sync_copy(idx_hbm.at[pl.ds(base, b_per_w)], idx_v)
        pltpu.async_copy(cb_hbm.at[idx_v], rows_v, sem).wait()
        pltpu.sync_copy(rows_v, out_hbm.at[pl.ds(base, b_per_w)])

    return gather_k(codebook, idx)


def kernel(z, codebook):
    z_p = jnp.transpose(z, (0, 2, 3, 1))
    z_flat = z_p.reshape(-1, DIM)
    idx3, onehot = _dist_argmin_onehot(z_flat, codebook)
    idx = idx3.reshape(NUM_TOKENS)
    zq_flat = _sc_gather(codebook, idx)
    z_q = zq_flat.reshape(z_p.shape)
    z_q = jnp.transpose(z_q, (0, 3, 1, 2))
    return (onehot, z_q)


# speedup vs baseline: 1.2679x; 1.2679x over previous
import functools

import jax
import jax.numpy as jnp
from jax import lax
from jax.experimental import pallas as pl
from jax.experimental.pallas import tpu as pltpu
from jax.experimental.pallas import tpu_sc as plsc

NUM_CODES = 8192
DIM = 256
NUM_TOKENS = 8192
TB = 256
NUM_TB = NUM_TOKENS // TB
CW = 128


def _dist_argmin_onehot_kernel(z_ref, cb_ref, idx_ref, oh_ref, csq_ref,
                               idsf_ref):
    @pl.when(pl.program_id(0) == 0)
    def _():
        cb0 = cb_ref[...]
        csq_ref[...] = jnp.sum(cb0 * cb0, axis=1)[None, :]
        idsf_ref[...] = lax.broadcasted_iota(
            jnp.int32, (1, NUM_CODES), 1).astype(jnp.float32)

    zc = z_ref[0].reshape(DIM, TB)
    z = zc.T
    cb = cb_ref[...]
    csq = csq_ref[...]
    zsq = jnp.sum(z * z, axis=1, keepdims=True)
    rmin = None
    rcol = None
    for c in range(NUM_CODES // CW):
        s = lax.dot_general(
            z, cb[c * CW:(c + 1) * CW, :], (((1,), (1,)), ((), ())),
            preferred_element_type=jnp.float32)
        dch = (zsq + csq[:, c * CW:(c + 1) * CW]) - 2.0 * s
        if c == 0:
            rmin = dch
            rcol = jnp.zeros((TB, CW), jnp.float32)
        else:
            upd = dch < rmin
            rmin = jnp.where(upd, dch, rmin)
            rcol = jnp.where(upd, float(c), rcol)
    m = jnp.min(rmin, axis=1, keepdims=True)
    ids = idsf_ref[...]
    lanef = ids[:, :CW]
    cand = jnp.where(rmin <= m, rcol * float(CW) + lanef, float(NUM_CODES))
    idxf = jnp.min(cand, axis=1)
    idx_ref[0, 0, :] = idxf.astype(jnp.int32)
    oh_ref[...] = (ids == idxf[:, None]).astype(jnp.float32)


def _dist_argmin_onehot(z_nchw, codebook):
    return pl.pallas_call(
        _dist_argmin_onehot_kernel,
        grid=(NUM_TB,),
        in_specs=[
            pl.BlockSpec((1, DIM, 8, 32), lambda i: (i // 4, 0, i % 4, 0)),
            pl.BlockSpec((NUM_CODES, DIM), lambda i: (0, 0)),
        ],
        out_specs=[
            pl.BlockSpec((1, 1, TB), lambda i: (i, 0, 0)),
            pl.BlockSpec((TB, NUM_CODES), lambda i: (i, 0)),
        ],
        out_shape=[
            jax.ShapeDtypeStruct((NUM_TB, 1, TB), jnp.int32),
            jax.ShapeDtypeStruct((NUM_TOKENS, NUM_CODES), jnp.float32),
        ],
        scratch_shapes=[pltpu.VMEM((1, NUM_CODES), jnp.float32),
                        pltpu.VMEM((1, NUM_CODES), jnp.float32)],
    )(z_nchw, codebook)


def _sc_gather(codebook, idx):
    info = plsc.get_sparse_core_info()
    nc, ns = info.num_cores, info.num_subcores
    nw = nc * ns
    b_per_w = NUM_TOKENS // nw
    mesh = plsc.VectorSubcoreMesh(core_axis_name="c", subcore_axis_name="s")

    @functools.partial(
        pl.kernel, mesh=mesh,
        out_type=jax.ShapeDtypeStruct((NUM_TOKENS, DIM), jnp.float32),
        scratch_types=[
            pltpu.VMEM((b_per_w,), jnp.int32),
            pltpu.VMEM((b_per_w, DIM), jnp.float32),
            pltpu.SemaphoreType.DMA,
        ],
    )
    def gather_k(cb_hbm, idx_hbm, out_hbm, idx_v, rows_v, sem):
        wid = lax.axis_index("s") * nc + lax.axis_index("c")
        base = wid * b_per_w
        pltpu.sync_copy(idx_hbm.at[pl.ds(base, b_per_w)], idx_v)
        pltpu.async_copy(cb_hbm.at[idx_v], rows_v, sem).wait()
        pltpu.sync_copy(rows_v, out_hbm.at[pl.ds(base, b_per_w)])

    return gather_k(codebook, idx)


def kernel(z, codebook):
    idx3, onehot = _dist_argmin_onehot(z, codebook)
    idx = idx3.reshape(NUM_TOKENS)
    zq_flat = _sc_gather(codebook, idx)
    z_q = zq_flat.reshape(8, 32, 32, DIM)
    z_q = jnp.transpose(z_q, (0, 3, 1, 2))
    return (onehot, z_q)


# speedup vs baseline: 1.6038x; 1.2650x over previous
import functools

import jax
import jax.numpy as jnp
from jax import lax
from jax.experimental import pallas as pl
from jax.experimental.pallas import tpu as pltpu
from jax.experimental.pallas import tpu_sc as plsc

NUM_CODES = 8192
DIM = 256
NUM_TOKENS = 8192
TB = 512
NUM_TB = NUM_TOKENS // TB
CW = 256


def _dist_argmin_onehot_kernel(z_ref, cb_ref, idx_ref, oh_ref, csq_ref,
                               idsf_ref):
    @pl.when(pl.program_id(0) == 0)
    def _():
        cb0 = cb_ref[...]
        csq_ref[...] = jnp.sum(cb0 * cb0, axis=1)[None, :]
        idsf_ref[...] = lax.broadcasted_iota(
            jnp.int32, (1, NUM_CODES), 1).astype(jnp.float32)

    z = z_ref[...]
    cb = cb_ref[...]
    csq = csq_ref[...]
    zsq = jnp.sum(z * z, axis=1, keepdims=True)
    rmin = None
    rcol = None
    for c in range(NUM_CODES // CW):
        s = lax.dot_general(
            z, cb[c * CW:(c + 1) * CW, :], (((1,), (1,)), ((), ())),
            preferred_element_type=jnp.float32)
        dch = (zsq + csq[:, c * CW:(c + 1) * CW]) - 2.0 * s
        if c == 0:
            rmin = dch
            rcol = jnp.zeros((TB, CW), jnp.float32)
        else:
            upd = dch < rmin
            rmin = jnp.where(upd, dch, rmin)
            rcol = jnp.where(upd, float(c), rcol)
    m = jnp.min(rmin, axis=1, keepdims=True)
    ids = idsf_ref[...]
    lanef = ids[:, :CW]
    cand = jnp.where(rmin <= m, rcol * float(CW) + lanef, float(NUM_CODES))
    idxf = jnp.min(cand, axis=1)
    idx_ref[0, 0, :] = idxf.astype(jnp.int32)
    oh_ref[...] = (ids == idxf[:, None]).astype(jnp.float32)


def _dist_argmin_onehot(z_flat, codebook):
    return pl.pallas_call(
        _dist_argmin_onehot_kernel,
        grid=(NUM_TB,),
        in_specs=[
            pl.BlockSpec((TB, DIM), lambda i: (i, 0)),
            pl.BlockSpec((NUM_CODES, DIM), lambda i: (0, 0)),
        ],
        out_specs=[
            pl.BlockSpec((1, 1, TB), lambda i: (i, 0, 0)),
            pl.BlockSpec((TB, NUM_CODES), lambda i: (i, 0)),
        ],
        out_shape=[
            jax.ShapeDtypeStruct((NUM_TB, 1, TB), jnp.int32),
            jax.ShapeDtypeStruct((NUM_TOKENS, NUM_CODES), jnp.float32),
        ],
        scratch_shapes=[pltpu.VMEM((1, NUM_CODES), jnp.float32),
                        pltpu.VMEM((1, NUM_CODES), jnp.float32)],
    )(z_flat, codebook)


def _sc_gather(codebook, idx):
    info = plsc.get_sparse_core_info()
    nc, ns = info.num_cores, info.num_subcores
    nw = nc * ns
    b_per_w = NUM_TOKENS // nw
    mesh = plsc.VectorSubcoreMesh(core_axis_name="c", subcore_axis_name="s")

    @functools.partial(
        pl.kernel, mesh=mesh,
        out_type=jax.ShapeDtypeStruct((NUM_TOKENS, DIM), jnp.float32),
        scratch_types=[
            pltpu.VMEM((b_per_w,), jnp.int32),
            pltpu.VMEM((b_per_w, DIM), jnp.float32),
            pltpu.SemaphoreType.DMA,
        ],
    )
    def gather_k(cb_hbm, idx_hbm, out_hbm, idx_v, rows_v, sem):
        wid = lax.axis_index("s") * nc + lax.axis_index("c")
        base = wid * b_per_w
        pltpu.sync_copy(idx_hbm.at[pl.ds(base, b_per_w)], idx_v)
        pltpu.async_copy(cb_hbm.at[idx_v], rows_v, sem).wait()
        pltpu.sync_copy(rows_v, out_hbm.at[pl.ds(base, b_per_w)])

    return gather_k(codebook, idx)


def kernel(z, codebook):
    z_p = jnp.transpose(z, (0, 2, 3, 1))
    z_flat = z_p.reshape(-1, DIM)
    idx3, onehot = _dist_argmin_onehot(z_flat, codebook)
    idx = idx3.reshape(NUM_TOKENS)
    zq_flat = _sc_gather(codebook, idx)
    z_q = zq_flat.reshape(z_p.shape)
    z_q = jnp.transpose(z_q, (0, 3, 1, 2))
    return (onehot, z_q)
